# speedup vs baseline: 1.5532x; 1.5532x over previous
import functools

import jax
import jax.numpy as jnp
from jax import lax
from jax.experimental import pallas as pl
from jax.experimental.pallas import tpu as pltpu
from jax.experimental.pallas import tpu_sc as plsc

_HIDDEN = 128
_CHUNK = 128


@functools.partial(jax.jit, static_argnames=("batch", "chunks_per_w", "nc", "ns"))
def _embed(x2d, weight, *, batch, chunks_per_w, nc, ns):
    mesh = plsc.VectorSubcoreMesh(core_axis_name="c", subcore_axis_name="s")

    @functools.partial(
        pl.kernel,
        mesh=mesh,
        out_type=jax.ShapeDtypeStruct((batch, _HIDDEN), jnp.float32),
        scratch_types=[
            pltpu.VMEM((chunks_per_w, _CHUNK), jnp.int32),
            pltpu.VMEM((chunks_per_w, _CHUNK, _HIDDEN), jnp.float32),
            pltpu.SemaphoreType.DMA,
            pltpu.SemaphoreType.DMA,
        ],
    )
    def k(idx_hbm, table_hbm, out_hbm, idx_v, rows_v, gsem, osem):
        wid = lax.axis_index("s") * nc + lax.axis_index("c")
        base = wid * chunks_per_w
        pltpu.sync_copy(idx_hbm.at[pl.ds(base, chunks_per_w)], idx_v)
        gathers = [
            pltpu.async_copy(table_hbm.at[idx_v.at[j]], rows_v.at[j], gsem)
            for j in range(chunks_per_w)
        ]
        stores = []
        for j in range(chunks_per_w):
            gathers[j].wait()
            stores.append(
                pltpu.async_copy(
                    rows_v.at[j], out_hbm.at[pl.ds((base + j) * _CHUNK, _CHUNK)], osem
                )
            )
        for s in stores:
            s.wait()

    return k(x2d, weight)


def kernel(x, weight):
    batch = x.shape[0]
    info = plsc.get_sparse_core_info()
    nc, ns = info.num_cores, info.num_subcores
    nw = nc * ns
    chunks_per_w = batch // (nw * _CHUNK)
    x2d = x.reshape(nw * chunks_per_w, _CHUNK).astype(jnp.int32)
    return _embed(x2d, weight, batch=batch, chunks_per_w=chunks_per_w, nc=nc, ns=ns)


# speedup vs baseline: 1.5758x; 1.0146x over previous
import functools

import jax
import jax.numpy as jnp
from jax import lax
from jax.experimental import pallas as pl
from jax.experimental.pallas import tpu as pltpu
from jax.experimental.pallas import tpu_sc as plsc

_HIDDEN = 128
_CHUNK = 128


@functools.partial(jax.jit, static_argnames=("batch", "chunks_per_w", "nc", "ns"))
def _embed(x2d, weight, *, batch, chunks_per_w, nc, ns):
    mesh = plsc.VectorSubcoreMesh(core_axis_name="c", subcore_axis_name="s")

    b_per_w = chunks_per_w * _CHUNK

    @functools.partial(
        pl.kernel,
        mesh=mesh,
        out_type=jax.ShapeDtypeStruct((batch, _HIDDEN), jnp.float32),
        scratch_types=[
            pltpu.VMEM((b_per_w,), jnp.int32),
            pltpu.VMEM((b_per_w, _HIDDEN), jnp.float32),
            pltpu.SemaphoreType.DMA,
        ],
    )
    def k(idx_hbm, table_hbm, out_hbm, idx_v, rows_v, gsem):
        wid = lax.axis_index("s") * nc + lax.axis_index("c")
        base = wid * b_per_w
        pltpu.sync_copy(idx_hbm.at[pl.ds(base, b_per_w)], idx_v)
        pltpu.async_copy(table_hbm.at[idx_v], rows_v, gsem).wait()
        pltpu.sync_copy(rows_v, out_hbm.at[pl.ds(base, b_per_w)])

    return k(x2d, weight)


def kernel(x, weight):
    batch = x.shape[0]
    info = plsc.get_sparse_core_info()
    nc, ns = info.num_cores, info.num_subcores
    nw = nc * ns
    chunks_per_w = batch // (nw * _CHUNK)
    return _embed(x.astype(jnp.int32), weight, batch=batch, chunks_per_w=chunks_per_w, nc=nc, ns=ns)
